# Initial kernel scaffold; baseline (speedup 1.0000x reference)
import functools

import jax
import jax.numpy as jnp
from jax import lax
from jax.experimental import pallas as pl
from jax.experimental.pallas import tpu as pltpu
from jax.experimental.pallas import tpu_sc as plsc

N = 10000
E = 320000
F_IN = 128
H = 512
G = 64
NCHUNK = 4
CH = H // NCHUNK
NC, NS = 2, 16
NW = NC * NS
KB = 80
RPT = N // NS
EPS = 1e-5

_mesh = plsc.VectorSubcoreMesh(core_axis_name="c", subcore_axis_name="s")


@functools.partial(
    pl.kernel,
    out_type=jax.ShapeDtypeStruct((NC, N, F_IN), jnp.float32),
    mesh=_mesh,
    scratch_types=[
        pltpu.VMEM((KB,), jnp.int32),
        pltpu.VMEM((KB,), jnp.int32),
        pltpu.VMEM((KB,), jnp.float32),
        pltpu.VMEM((KB, F_IN), jnp.float32),
        pltpu.VMEM_SHARED((N, F_IN), jnp.float32),
        pltpu.SemaphoreType.DMA,
    ],
)
def _sc_aggr1(x_hbm, src_hbm, dst_hbm, ew_hbm, zeros_hbm, out_hbm,
              sidx, didx, ew_v, rows, acc, sem):
    c = lax.axis_index("c")
    s = lax.axis_index("s")
    w = c * NS + s
    r0 = s * RPT
    pltpu.sync_copy(zeros_hbm, acc.at[pl.ds(r0, RPT)])
    plsc.subcore_barrier()
    epw = E // NW
    base = w * epw

    @pl.loop(0, epw // KB)
    def _batch(j):
        off = base + j * KB
        pltpu.sync_copy(src_hbm.at[pl.ds(off, KB)], sidx)
        pltpu.sync_copy(dst_hbm.at[pl.ds(off, KB)], didx)
        pltpu.sync_copy(ew_hbm.at[pl.ds(off, KB)], ew_v)
        pltpu.async_copy(x_hbm.at[sidx], rows, sem).wait()

        @pl.loop(0, KB)
        def _edge(e):
            wgt = ew_v[e]
            for i in range(F_IN // 16):
                sl = pl.ds(i * 16, 16)
                rows[e, sl] = rows[e, sl] * wgt

        pltpu.sync_copy(rows, acc.at[didx], add=True)

    plsc.subcore_barrier()
    pltpu.sync_copy(acc.at[pl.ds(r0, RPT)], out_hbm.at[c, pl.ds(r0, RPT)])


@functools.partial(
    pl.kernel,
    out_type=jax.ShapeDtypeStruct((E,), jnp.float32),
    mesh=_mesh,
    scratch_types=[
        pltpu.VMEM((KB,), jnp.int32),
        pltpu.VMEM((KB,), jnp.int32),
        pltpu.VMEM((KB, H), jnp.float32),
        pltpu.VMEM((KB, H), jnp.float32),
        pltpu.VMEM((KB,), jnp.float32),
        pltpu.SemaphoreType.DMA,
        pltpu.SemaphoreType.DMA,
    ],
)
def _sc_ew2(h_hbm, src_hbm, dst_hbm, out_hbm,
            sidx, didx, srows, drows, tot, sem1, sem2):
    c = lax.axis_index("c")
    s = lax.axis_index("s")
    w = c * NS + s
    epw = E // NW
    base = w * epw

    @pl.loop(0, epw // KB)
    def _batch(j):
        off = base + j * KB
        pltpu.sync_copy(src_hbm.at[pl.ds(off, KB)], sidx)
        pltpu.sync_copy(dst_hbm.at[pl.ds(off, KB)], didx)
        cp1 = pltpu.async_copy(h_hbm.at[sidx], srows, sem1)
        cp2 = pltpu.async_copy(h_hbm.at[didx], drows, sem2)
        cp1.wait()
        cp2.wait()

        @pl.loop(0, KB)
        def _edge(e):
            accs = [jnp.zeros((16,), jnp.float32) for _ in range(4)]
            for i in range(H // 16):
                sl = pl.ds(i * 16, 16)
                d = srows[e, sl] - drows[e, sl]
                accs[i % 4] = accs[i % 4] + d * d
            tot[e] = jnp.sum(accs[0] + accs[1] + accs[2] + accs[3])

        for b in range(KB // 16):
            sl = pl.ds(b * 16, 16)
            tot[sl] = jnp.exp(tot[sl] * (-1.0 / H))
        pltpu.sync_copy(tot, out_hbm.at[pl.ds(off, KB)])


@functools.partial(
    pl.kernel,
    out_type=jax.ShapeDtypeStruct((NCHUNK, N, CH), jnp.float32),
    mesh=_mesh,
    scratch_types=[
        pltpu.VMEM((KB,), jnp.int32),
        pltpu.VMEM((KB,), jnp.int32),
        pltpu.VMEM((KB,), jnp.float32),
        pltpu.VMEM((KB, CH), jnp.float32),
        pltpu.VMEM_SHARED((N, CH), jnp.float32),
        pltpu.SemaphoreType.DMA,
    ],
)
def _sc_aggr2(hs_hbm, src4_hbm, dst_hbm, ew2_hbm, zeros_hbm, out_hbm,
              sidx, didx, ew_v, rows, acc, sem):
    c = lax.axis_index("c")
    s = lax.axis_index("s")
    r0 = s * RPT
    ept = E // NS
    base = s * ept
    for p in range(NCHUNK // NC):
        chunk = p * NC + c
        pltpu.sync_copy(zeros_hbm, acc.at[pl.ds(r0, RPT)])
        plsc.subcore_barrier()

        @pl.loop(0, ept // KB)
        def _batch(j):
            off = base + j * KB
            pltpu.sync_copy(src4_hbm.at[chunk, pl.ds(off, KB)], sidx)
            pltpu.sync_copy(dst_hbm.at[pl.ds(off, KB)], didx)
            pltpu.sync_copy(ew2_hbm.at[pl.ds(off, KB)], ew_v)
            pltpu.async_copy(hs_hbm.at[sidx], rows, sem).wait()

            @pl.loop(0, KB)
            def _edge(e):
                wgt = ew_v[e]
                for i in range(CH // 16):
                    sl = pl.ds(i * 16, 16)
                    rows[e, sl] = rows[e, sl] * wgt

            pltpu.sync_copy(rows, acc.at[didx], add=True)

        plsc.subcore_barrier()
        pltpu.sync_copy(acc.at[pl.ds(r0, RPT)], out_hbm.at[chunk, pl.ds(r0, RPT)])
        plsc.subcore_barrier()


def _tc_layer1_body(x_ref, p_ref, wrel_ref, wroot_ref, b_ref, g_ref, be_ref,
                    batch_ref, h_ref, x1_ref):
    aggr = p_ref[0] + p_ref[1]
    dn = (((1,), (1,)), ((), ()))
    hp = lax.dot_general(aggr, wrel_ref[...], dn,
                         preferred_element_type=jnp.float32)
    hp = hp + lax.dot_general(x_ref[...], wroot_ref[...], dn,
                              preferred_element_type=jnp.float32)
    hp = hp + b_ref[...][None, :]
    mean = jnp.mean(hp, axis=0)
    var = jnp.mean((hp - mean[None, :]) ** 2, axis=0)
    scale = lax.rsqrt(var + EPS) * g_ref[...]
    hn = (hp - mean[None, :]) * scale[None, :] + be_ref[...][None, :]
    h_ref[...] = hn
    b = batch_ref[...]
    P = (lax.broadcasted_iota(jnp.int32, (G, N), 0) == b[None, :]
         ).astype(jnp.float32)
    counts = jnp.clip(jnp.sum(P, axis=1), 1.0, None)
    x1_ref[...] = jnp.dot(P, hn, preferred_element_type=jnp.float32) \
        / counts[:, None]


def _tc_layer1(x, p, wrel, wroot, b, g, be, batch):
    return pl.pallas_call(
        _tc_layer1_body,
        out_shape=(jax.ShapeDtypeStruct((N, H), jnp.float32),
                   jax.ShapeDtypeStruct((G, H), jnp.float32)),
    )(x, p, wrel, wroot, b, g, be, batch)


def _tc_layer2_body(h_ref, a2_ref, wrel_ref, wroot_ref, b_ref, g_ref, be_ref,
                    batch_ref, x1_ref, out_ref):
    dn = (((1,), (1,)), ((), ()))
    h2 = lax.dot_general(a2_ref[...], wrel_ref[...], dn,
                         preferred_element_type=jnp.float32)
    h2 = h2 + lax.dot_general(h_ref[...], wroot_ref[...], dn,
                              preferred_element_type=jnp.float32)
    h2 = h2 + b_ref[...][None, :]
    mean = jnp.mean(h2, axis=0)
    var = jnp.mean((h2 - mean[None, :]) ** 2, axis=0)
    b = batch_ref[...]
    P = (lax.broadcasted_iota(jnp.int32, (G, N), 0) == b[None, :]
         ).astype(jnp.float32)
    counts = jnp.clip(jnp.sum(P, axis=1), 1.0, None)
    seg = jnp.dot(P, h2, preferred_element_type=jnp.float32) / counts[:, None]
    scale = lax.rsqrt(var + EPS) * g_ref[...]
    out_ref[...] = (seg - mean[None, :]) * scale[None, :] \
        + be_ref[...][None, :] + x1_ref[...]


def _tc_layer2(h, a2, wrel, wroot, b, g, be, batch, x1):
    return pl.pallas_call(
        _tc_layer2_body,
        out_shape=jax.ShapeDtypeStruct((G, H), jnp.float32),
    )(h, a2, wrel, wroot, b, g, be, batch, x1)


def kernel(x, edge_index, edge_weight, batch, W1_rel, W1_root, b1, gamma1,
           beta1, W2_rel, W2_root, b2, gamma2, beta2):
    src = edge_index[0]
    dst = edge_index[1]
    zeros = jnp.zeros((RPT, F_IN), jnp.float32)

    p1 = _sc_aggr1(x, src, dst, edge_weight, zeros)
    h, x1 = _tc_layer1(x, p1, W1_rel, W1_root, b1, gamma1, beta1, batch)
    ew2 = _sc_ew2(h, src, dst)

    hs = h.reshape(N, NCHUNK, CH).transpose(1, 0, 2).reshape(NCHUNK * N, CH)
    src4 = src[None, :] + (jnp.arange(NCHUNK, dtype=jnp.int32) * N)[:, None]
    a2c = _sc_aggr2(hs, src4, dst, ew2, zeros)
    aggr2 = a2c.transpose(1, 0, 2).reshape(N, H)

    return _tc_layer2(h, aggr2, W2_rel, W2_root, b2, gamma2, beta2, batch, x1)


# speedup vs baseline: 1.4258x; 1.4258x over previous
import functools

import jax
import jax.numpy as jnp
from jax import lax
from jax.experimental import pallas as pl
from jax.experimental.pallas import tpu as pltpu
from jax.experimental.pallas import tpu_sc as plsc

N = 10000
E = 320000
F_IN = 128
H = 512
G = 64
NCHUNK = 4
CH = H // NCHUNK
NC, NS = 2, 16
NW = NC * NS
KB = 80
NP = 10240
RPT = NP // NS
EPS = 1e-5

_mesh = plsc.VectorSubcoreMesh(core_axis_name="c", subcore_axis_name="s",
                               num_cores=NC, num_subcores=NS)

_GDN = lax.GatherDimensionNumbers(offset_dims=(), collapsed_slice_dims=(0,),
                                  start_index_map=(0,))


def _shuffle(v, idx):
    return lax.gather(v, idx[:, None], _GDN, (1,),
                      mode=lax.GatherScatterMode.PROMISE_IN_BOUNDS)


@functools.partial(
    pl.kernel,
    out_type=jax.ShapeDtypeStruct((NC, NP, F_IN), jnp.float32),
    mesh=_mesh,
    scratch_types=[
        pltpu.VMEM((KB,), jnp.int32),
        pltpu.VMEM((KB,), jnp.int32),
        pltpu.VMEM((KB,), jnp.float32),
        pltpu.VMEM((KB, F_IN), jnp.float32),
        pltpu.VMEM_SHARED((NP, F_IN), jnp.float32),
        pltpu.SemaphoreType.DMA,
    ],
)
def _sc_aggr1(x_hbm, src_hbm, dst_hbm, ew_hbm, zeros_hbm, out_hbm,
              sidx, didx, ew_v, rows, acc, sem):
    c = lax.axis_index("c")
    s = lax.axis_index("s")
    w = c * NS + s
    r0 = s * RPT
    pltpu.sync_copy(zeros_hbm, acc.at[pl.ds(r0, RPT)])
    plsc.subcore_barrier()
    epw = E // NW
    base = w * epw

    @pl.loop(0, epw // KB)
    def _batch(j):
        off = base + j * KB
        pltpu.sync_copy(src_hbm.at[pl.ds(off, KB)], sidx)
        pltpu.sync_copy(dst_hbm.at[pl.ds(off, KB)], didx)
        pltpu.sync_copy(ew_hbm.at[pl.ds(off, KB)], ew_v)
        pltpu.async_copy(x_hbm.at[sidx], rows, sem).wait()

        @pl.loop(0, KB // 16)
        def _grp(g):
            w16 = ew_v[pl.ds(g * 16, 16)]
            for e in range(16):
                we = w16[e]
                ei = g * 16 + e
                for i in range(F_IN // 16):
                    sl = pl.ds(i * 16, 16)
                    rows[ei, sl] = rows[ei, sl] * we

        pltpu.sync_copy(rows, acc.at[didx], add=True)

    plsc.subcore_barrier()
    pltpu.sync_copy(acc.at[pl.ds(r0, RPT)], out_hbm.at[c, pl.ds(r0, RPT)])


@functools.partial(
    pl.kernel,
    out_type=jax.ShapeDtypeStruct((E,), jnp.float32),
    mesh=_mesh,
    scratch_types=[
        pltpu.VMEM((KB,), jnp.int32),
        pltpu.VMEM((KB,), jnp.int32),
        pltpu.VMEM((KB, H), jnp.float32),
        pltpu.VMEM((KB, H), jnp.float32),
        pltpu.VMEM((KB,), jnp.float32),
        pltpu.SemaphoreType.DMA,
        pltpu.SemaphoreType.DMA,
    ],
)
def _sc_ew2(h_hbm, src_hbm, dst_hbm, out_hbm,
            sidx, didx, srows, drows, tot, sem1, sem2):
    c = lax.axis_index("c")
    s = lax.axis_index("s")
    w = c * NS + s
    epw = E // NW
    base = w * epw

    @pl.loop(0, epw // KB)
    def _batch(j):
        off = base + j * KB
        pltpu.sync_copy(src_hbm.at[pl.ds(off, KB)], sidx)
        pltpu.sync_copy(dst_hbm.at[pl.ds(off, KB)], didx)
        cp1 = pltpu.async_copy(h_hbm.at[sidx], srows, sem1)
        cp2 = pltpu.async_copy(h_hbm.at[didx], drows, sem2)
        cp1.wait()
        cp2.wait()

        @pl.loop(0, KB // 16)
        def _grp(g):
            lane = lax.broadcasted_iota(jnp.int32, (16,), 0)
            tvec = jnp.zeros((16,), jnp.float32)
            for e in range(16):
                ei = g * 16 + e
                accs = [jnp.zeros((16,), jnp.float32) for _ in range(4)]
                for i in range(H // 16):
                    sl = pl.ds(i * 16, 16)
                    d = srows[ei, sl] - drows[ei, sl]
                    accs[i % 4] = accs[i % 4] + d * d
                a = (accs[0] + accs[1]) + (accs[2] + accs[3])
                for m in (8, 4, 2, 1):
                    a = a + _shuffle(a, lane ^ m)
                tvec = jnp.where(lane == e, a, tvec)
            tot[pl.ds(g * 16, 16)] = jnp.exp(tvec * (-1.0 / H))

        pltpu.sync_copy(tot, out_hbm.at[pl.ds(off, KB)])


@functools.partial(
    pl.kernel,
    out_type=jax.ShapeDtypeStruct((NCHUNK, NP, CH), jnp.float32),
    mesh=_mesh,
    scratch_types=[
        pltpu.VMEM((KB,), jnp.int32),
        pltpu.VMEM((KB,), jnp.int32),
        pltpu.VMEM((KB,), jnp.float32),
        pltpu.VMEM((KB, CH), jnp.float32),
        pltpu.VMEM_SHARED((NP, CH), jnp.float32),
        pltpu.SemaphoreType.DMA,
    ],
)
def _sc_aggr2(hs_hbm, src4_hbm, dst_hbm, ew2_hbm, zeros_hbm, out_hbm,
              sidx, didx, ew_v, rows, acc, sem):
    c = lax.axis_index("c")
    s = lax.axis_index("s")
    r0 = s * RPT
    ept = E // NS
    base = s * ept
    for p in range(NCHUNK // NC):
        chunk = p * NC + c
        pltpu.sync_copy(zeros_hbm, acc.at[pl.ds(r0, RPT)])
        plsc.subcore_barrier()

        @pl.loop(0, ept // KB)
        def _batch(j):
            off = base + j * KB
            pltpu.sync_copy(src4_hbm.at[pl.ds(chunk * E + off, KB)], sidx)
            pltpu.sync_copy(dst_hbm.at[pl.ds(off, KB)], didx)
            pltpu.sync_copy(ew2_hbm.at[pl.ds(off, KB)], ew_v)
            pltpu.async_copy(hs_hbm.at[sidx], rows, sem).wait()

            @pl.loop(0, KB // 16)
            def _grp(g):
                w16 = ew_v[pl.ds(g * 16, 16)]
                for e in range(16):
                    we = w16[e]
                    ei = g * 16 + e
                    for i in range(CH // 16):
                        sl = pl.ds(i * 16, 16)
                        rows[ei, sl] = rows[ei, sl] * we

            pltpu.sync_copy(rows, acc.at[didx], add=True)

        plsc.subcore_barrier()
        pltpu.sync_copy(acc.at[pl.ds(r0, RPT)], out_hbm.at[chunk, pl.ds(r0, RPT)])
        plsc.subcore_barrier()


NB = 8
BR = NP // NB


def _tc1a_body(x_ref, p_ref, wrel_ref, wroot_ref, b_ref,
               hp_ref, sum_ref, ssq_ref, sacc, qacc):
    i = pl.program_id(0)
    rid = i * BR + lax.broadcasted_iota(jnp.int32, (BR, 1), 0)
    valid = (rid < N).astype(jnp.float32)
    aggr = p_ref[0] + p_ref[1]
    dn = (((1,), (1,)), ((), ()))
    hp = lax.dot_general(aggr, wrel_ref[...], dn,
                         preferred_element_type=jnp.float32)
    hp = hp + lax.dot_general(x_ref[...], wroot_ref[...], dn,
                              preferred_element_type=jnp.float32)
    hp = hp + b_ref[...][None, :]
    hp_ref[...] = hp

    @pl.when(i == 0)
    def _init():
        sacc[...] = jnp.zeros_like(sacc)
        qacc[...] = jnp.zeros_like(qacc)

    hpm = hp * valid
    sacc[...] += jnp.sum(hpm, axis=0, keepdims=True)
    qacc[...] += jnp.sum(hpm * hp, axis=0, keepdims=True)

    @pl.when(i == NB - 1)
    def _fin():
        sum_ref[...] = sacc[...]
        ssq_ref[...] = qacc[...]


def _tc1a(x, p, wrel, wroot, b):
    return pl.pallas_call(
        _tc1a_body,
        grid=(NB,),
        in_specs=[
            pl.BlockSpec((BR, F_IN), lambda i: (i, 0)),
            pl.BlockSpec((NC, BR, F_IN), lambda i: (0, i, 0)),
            pl.BlockSpec((H, F_IN), lambda i: (0, 0)),
            pl.BlockSpec((H, F_IN), lambda i: (0, 0)),
            pl.BlockSpec((H,), lambda i: (0,)),
        ],
        out_specs=[
            pl.BlockSpec((BR, H), lambda i: (i, 0)),
            pl.BlockSpec((1, H), lambda i: (0, 0)),
            pl.BlockSpec((1, H), lambda i: (0, 0)),
        ],
        out_shape=[
            jax.ShapeDtypeStruct((NP, H), jnp.float32),
            jax.ShapeDtypeStruct((1, H), jnp.float32),
            jax.ShapeDtypeStruct((1, H), jnp.float32),
        ],
        scratch_shapes=[
            pltpu.VMEM((1, H), jnp.float32),
            pltpu.VMEM((1, H), jnp.float32),
        ],
    )(x, p, wrel, wroot, b)


def _tc1b_body(hp_ref, sum_ref, ssq_ref, g_ref, be_ref, batch_ref,
               h_ref, x1_ref, seg_acc, cnt_acc):
    i = pl.program_id(0)
    mean = sum_ref[...] * (1.0 / N)
    var = ssq_ref[...] * (1.0 / N) - mean * mean
    scale = lax.rsqrt(var + EPS) * g_ref[...][None, :]
    hn = (hp_ref[...] - mean) * scale + be_ref[...][None, :]
    h_ref[...] = hn
    PT = (lax.broadcasted_iota(jnp.int32, (BR, G), 1) == batch_ref[...]
          ).astype(jnp.float32)
    dnt = (((0,), (0,)), ((), ()))

    @pl.when(i == 0)
    def _init():
        seg_acc[...] = jnp.zeros_like(seg_acc)
        cnt_acc[...] = jnp.zeros_like(cnt_acc)

    seg_acc[...] += lax.dot_general(PT, hn, dnt,
                                    preferred_element_type=jnp.float32)
    cnt_acc[...] += lax.dot_general(PT, jnp.ones((BR, 1), jnp.float32), dnt,
                                    preferred_element_type=jnp.float32)

    @pl.when(i == NB - 1)
    def _fin():
        counts = jnp.clip(cnt_acc[...], 1.0, None)
        x1_ref[...] = seg_acc[...] / counts


def _tc1b(hp, ssum, ssq, g, be, batch):
    return pl.pallas_call(
        _tc1b_body,
        grid=(NB,),
        in_specs=[
            pl.BlockSpec((BR, H), lambda i: (i, 0)),
            pl.BlockSpec((1, H), lambda i: (0, 0)),
            pl.BlockSpec((1, H), lambda i: (0, 0)),
            pl.BlockSpec((H,), lambda i: (0,)),
            pl.BlockSpec((H,), lambda i: (0,)),
            pl.BlockSpec((BR, 1), lambda i: (i, 0)),
        ],
        out_specs=[
            pl.BlockSpec((BR, H), lambda i: (i, 0)),
            pl.BlockSpec((G, H), lambda i: (0, 0)),
        ],
        out_shape=[
            jax.ShapeDtypeStruct((NP, H), jnp.float32),
            jax.ShapeDtypeStruct((G, H), jnp.float32),
        ],
        scratch_shapes=[
            pltpu.VMEM((G, H), jnp.float32),
            pltpu.VMEM((G, 1), jnp.float32),
        ],
    )(hp, ssum, ssq, g, be, batch)


def _tc2_body(h_ref, a2_ref, wrel_ref, wroot_ref, b_ref, g_ref, be_ref,
              batch_ref, x1_ref, out_ref, sacc, qacc, seg_acc, cnt_acc):
    i = pl.program_id(0)
    rid = i * BR + lax.broadcasted_iota(jnp.int32, (BR, 1), 0)
    valid = (rid < N).astype(jnp.float32)
    dn = (((1,), (1,)), ((), ()))
    h2 = lax.dot_general(a2_ref[...], wrel_ref[...], dn,
                         preferred_element_type=jnp.float32)
    h2 = h2 + lax.dot_general(h_ref[...], wroot_ref[...], dn,
                              preferred_element_type=jnp.float32)
    h2 = h2 + b_ref[...][None, :]
    PT = (lax.broadcasted_iota(jnp.int32, (BR, G), 1) == batch_ref[...]
          ).astype(jnp.float32)
    dnt = (((0,), (0,)), ((), ()))

    @pl.when(i == 0)
    def _init():
        sacc[...] = jnp.zeros_like(sacc)
        qacc[...] = jnp.zeros_like(qacc)
        seg_acc[...] = jnp.zeros_like(seg_acc)
        cnt_acc[...] = jnp.zeros_like(cnt_acc)

    h2m = h2 * valid
    sacc[...] += jnp.sum(h2m, axis=0, keepdims=True)
    qacc[...] += jnp.sum(h2m * h2, axis=0, keepdims=True)
    seg_acc[...] += lax.dot_general(PT, h2, dnt,
                                    preferred_element_type=jnp.float32)
    cnt_acc[...] += lax.dot_general(PT, jnp.ones((BR, 1), jnp.float32), dnt,
                                    preferred_element_type=jnp.float32)

    @pl.when(i == NB - 1)
    def _fin():
        mean = sacc[...] * (1.0 / N)
        var = qacc[...] * (1.0 / N) - mean * mean
        scale = lax.rsqrt(var + EPS) * g_ref[...][None, :]
        counts = jnp.clip(cnt_acc[...], 1.0, None)
        seg = seg_acc[...] / counts
        out_ref[...] = (seg - mean) * scale + be_ref[...][None, :] \
            + x1_ref[...]


def _tc_layer2(h, a2, wrel, wroot, b, g, be, batch, x1):
    return pl.pallas_call(
        _tc2_body,
        grid=(NB,),
        in_specs=[
            pl.BlockSpec((BR, H), lambda i: (i, 0)),
            pl.BlockSpec((BR, H), lambda i: (i, 0)),
            pl.BlockSpec((H, H), lambda i: (0, 0)),
            pl.BlockSpec((H, H), lambda i: (0, 0)),
            pl.BlockSpec((H,), lambda i: (0,)),
            pl.BlockSpec((H,), lambda i: (0,)),
            pl.BlockSpec((H,), lambda i: (0,)),
            pl.BlockSpec((BR, 1), lambda i: (i, 0)),
            pl.BlockSpec((G, H), lambda i: (0, 0)),
        ],
        out_specs=pl.BlockSpec((G, H), lambda i: (0, 0)),
        out_shape=jax.ShapeDtypeStruct((G, H), jnp.float32),
        scratch_shapes=[
            pltpu.VMEM((1, H), jnp.float32),
            pltpu.VMEM((1, H), jnp.float32),
            pltpu.VMEM((G, H), jnp.float32),
            pltpu.VMEM((G, 1), jnp.float32),
        ],
    )(h, a2, wrel, wroot, b, g, be, batch, x1)


def kernel(x, edge_index, edge_weight, batch, W1_rel, W1_root, b1, gamma1,
           beta1, W2_rel, W2_root, b2, gamma2, beta2):
    src = edge_index[0]
    dst = edge_index[1]
    zeros = jnp.zeros((RPT, F_IN), jnp.float32)

    x_pad = jnp.pad(x, ((0, NP - N), (0, 0)))
    batch_pad = jnp.pad(batch, (0, NP - N), constant_values=G)[:, None]

    p1 = _sc_aggr1(x, src, dst, edge_weight, zeros)
    hp, ssum, ssq = _tc1a(x_pad, p1, W1_rel, W1_root, b1)
    h_pad, x1 = _tc1b(hp, ssum, ssq, gamma1, beta1, batch_pad)
    h = h_pad[:N]
    ew2 = _sc_ew2(h, src, dst)

    hs = h.reshape(N, NCHUNK, CH).transpose(1, 0, 2).reshape(NCHUNK * N, CH)
    src4 = (src[None, :]
            + (jnp.arange(NCHUNK, dtype=jnp.int32) * N)[:, None]).reshape(-1)
    a2c = _sc_aggr2(hs, src4, dst, ew2, zeros)
    aggr2_pad = a2c.transpose(1, 0, 2).reshape(NP, H)

    return _tc_layer2(h_pad, aggr2_pad, W2_rel, W2_root, b2, gamma2, beta2,
                      batch_pad, x1)


# speedup vs baseline: 4.8373x; 3.3927x over previous
import functools

import jax
import jax.numpy as jnp
from jax import lax
from jax.experimental import pallas as pl
from jax.experimental.pallas import tpu as pltpu
from jax.experimental.pallas import tpu_sc as plsc

N = 10000
E = 320000
F_IN = 128
H = 512
G = 64
NCHUNK = 4
CH = H // NCHUNK
NC, NS = 2, 16
NW = NC * NS
KB = 80
NB1 = E // NW // KB
NB2 = E // NS // KB
NBC = 25
NCH1 = NB1 // NBC
NCH2 = NB2 // NBC
NP = 10240
RPT = NP // NS
EPS = 1e-5

_mesh = plsc.VectorSubcoreMesh(core_axis_name="c", subcore_axis_name="s",
                               num_cores=NC, num_subcores=NS)

_GDN = lax.GatherDimensionNumbers(offset_dims=(), collapsed_slice_dims=(0,),
                                  start_index_map=(0,))


def _shuffle(v, idx):
    return lax.gather(v, idx[:, None], _GDN, (1,),
                      mode=lax.GatherScatterMode.PROMISE_IN_BOUNDS)


def _scale_rows(buf, wrow, width):
    @pl.loop(0, KB // 16)
    def _grp(g):
        w16 = wrow[pl.ds(g * 16, 16)]
        for e in range(16):
            we = w16[e]
            ei = g * 16 + e
            for i in range(width // 16):
                sl = pl.ds(i * 16, 16)
                buf[ei, sl] = buf[ei, sl] * we


@functools.partial(
    pl.kernel,
    out_type=jax.ShapeDtypeStruct((NC, NP, F_IN), jnp.float32),
    mesh=_mesh,
    scratch_types=[
        pltpu.VMEM((NBC, KB), jnp.int32),
        pltpu.VMEM((NBC, KB), jnp.int32),
        pltpu.VMEM((NBC, KB), jnp.float32),
        pltpu.VMEM((KB, F_IN), jnp.float32),
        pltpu.VMEM((KB, F_IN), jnp.float32),
        pltpu.VMEM_SHARED((NP, F_IN), jnp.float32),
        pltpu.SemaphoreType.DMA,
        pltpu.SemaphoreType.DMA,
    ],
)
def _sc_aggr1(x_hbm, src_hbm, dst_hbm, ew_hbm, zeros_hbm, out_hbm,
              sidx, didx, ew2d, rows0, rows1, acc, sem0, sem1):
    c = lax.axis_index("c")
    s = lax.axis_index("s")
    w = c * NS + s
    r0 = s * RPT
    pltpu.sync_copy(zeros_hbm, acc.at[pl.ds(r0, RPT)])
    plsc.subcore_barrier()

    def _proc(j, buf, sem):
        pltpu.make_async_copy(x_hbm.at[pl.ds(0, KB)], buf, sem).wait()
        _scale_rows(buf, ew2d.at[j], F_IN)
        pltpu.sync_copy(buf, acc.at[didx.at[j]], add=True)

    @pl.loop(0, NCH1)
    def _tc(tc):
        pltpu.sync_copy(src_hbm.at[w, tc], sidx)
        pltpu.sync_copy(dst_hbm.at[w, tc], didx)
        pltpu.sync_copy(ew_hbm.at[w, tc], ew2d)
        pltpu.async_copy(x_hbm.at[sidx.at[0]], rows0, sem0)

        @pl.loop(0, NBC // 2)
        def _jj(jj):
            j0 = jj * 2
            pltpu.async_copy(x_hbm.at[sidx.at[j0 + 1]], rows1, sem1)
            _proc(j0, rows0, sem0)
            pltpu.async_copy(x_hbm.at[sidx.at[j0 + 2]], rows0, sem0)
            _proc(j0 + 1, rows1, sem1)

        _proc(NBC - 1, rows0, sem0)

    plsc.subcore_barrier()
    pltpu.sync_copy(acc.at[pl.ds(r0, RPT)], out_hbm.at[c, pl.ds(r0, RPT)])


@functools.partial(
    pl.kernel,
    out_type=jax.ShapeDtypeStruct((NW, NCH1, NBC, KB), jnp.float32),
    mesh=_mesh,
    scratch_types=[
        pltpu.VMEM((NBC, KB), jnp.int32),
        pltpu.VMEM((NBC, KB), jnp.int32),
        pltpu.VMEM((KB, H // 2), jnp.int32),
        pltpu.VMEM((KB, H // 2), jnp.int32),
        pltpu.VMEM((KB, H // 2), jnp.int32),
        pltpu.VMEM((KB, H // 2), jnp.int32),
        pltpu.VMEM((NBC, KB), jnp.float32),
        pltpu.SemaphoreType.DMA,
        pltpu.SemaphoreType.DMA,
        pltpu.SemaphoreType.DMA,
        pltpu.SemaphoreType.DMA,
    ],
)
def _sc_ew2(hbf_hbm, src_hbm, dst_hbm, out_hbm,
            sidx, didx, sr0, dr0, sr1, dr1, tot,
            ss0, sd0, ss1, sd1):
    c = lax.axis_index("c")
    s = lax.axis_index("s")
    w = c * NS + s

    def _issue(j, srows, drows, sems, semd):
        pltpu.async_copy(hbf_hbm.at[sidx.at[j]], srows, sems)
        pltpu.async_copy(hbf_hbm.at[didx.at[j]], drows, semd)

    def _proc(j, srows, drows, sems, semd):
        pltpu.make_async_copy(hbf_hbm.at[pl.ds(0, KB)], srows, sems).wait()
        pltpu.make_async_copy(hbf_hbm.at[pl.ds(0, KB)], drows, semd).wait()

        @pl.loop(0, KB // 16)
        def _grp(g):
            lane = lax.broadcasted_iota(jnp.int32, (16,), 0)
            tvec = jnp.zeros((16,), jnp.float32)
            msk = jnp.int32(-65536)
            for e in range(16):
                ei = g * 16 + e
                accs = [jnp.zeros((16,), jnp.float32) for _ in range(4)]
                for t in range(H // 32):
                    sl = pl.ds(t * 16, 16)
                    siv = srows[ei, sl]
                    div = drows[ei, sl]
                    qh = lax.bitcast_convert_type(siv & msk, jnp.float32) \
                        - lax.bitcast_convert_type(div & msk, jnp.float32)
                    ql = lax.bitcast_convert_type(siv << 16, jnp.float32) \
                        - lax.bitcast_convert_type(div << 16, jnp.float32)
                    k = 2 * (t % 2)
                    accs[k] = accs[k] + qh * qh
                    accs[k + 1] = accs[k + 1] + ql * ql
                a = (accs[0] + accs[1]) + (accs[2] + accs[3])
                for m in (8, 4, 2, 1):
                    a = a + _shuffle(a, lane ^ m)
                tvec = jnp.where(lane == e, a, tvec)
            tot[j, pl.ds(g * 16, 16)] = jnp.exp(tvec * (-1.0 / H))

    @pl.loop(0, NCH1)
    def _tc(tc):
        pltpu.sync_copy(src_hbm.at[w, tc], sidx)
        pltpu.sync_copy(dst_hbm.at[w, tc], didx)
        _issue(0, sr0, dr0, ss0, sd0)

        @pl.loop(0, NBC // 2)
        def _jj(jj):
            j0 = jj * 2
            _issue(j0 + 1, sr1, dr1, ss1, sd1)
            _proc(j0, sr0, dr0, ss0, sd0)
            _issue(j0 + 2, sr0, dr0, ss0, sd0)
            _proc(j0 + 1, sr1, dr1, ss1, sd1)

        _proc(NBC - 1, sr0, dr0, ss0, sd0)
        pltpu.sync_copy(tot, out_hbm.at[w, tc])


@functools.partial(
    pl.kernel,
    out_type=jax.ShapeDtypeStruct((NCHUNK, NP, CH), jnp.float32),
    mesh=_mesh,
    scratch_types=[
        pltpu.VMEM((NBC, KB), jnp.int32),
        pltpu.VMEM((NBC, KB), jnp.int32),
        pltpu.VMEM((NBC, KB), jnp.float32),
        pltpu.VMEM((KB, CH), jnp.float32),
        pltpu.VMEM((KB, CH), jnp.float32),
        pltpu.VMEM_SHARED((NP, CH), jnp.float32),
        pltpu.SemaphoreType.DMA,
        pltpu.SemaphoreType.DMA,
    ],
)
def _sc_aggr2(hs_hbm, src4_hbm, dst_hbm, ew2_hbm, zeros_hbm, out_hbm,
              sidx, didx, ew2d, rows0, rows1, acc, sem0, sem1):
    c = lax.axis_index("c")
    s = lax.axis_index("s")
    r0 = s * RPT

    def _proc(j, buf, sem):
        pltpu.make_async_copy(hs_hbm.at[pl.ds(0, KB)], buf, sem).wait()
        _scale_rows(buf, ew2d.at[j], CH)
        pltpu.sync_copy(buf, acc.at[didx.at[j]], add=True)

    for p in range(NCHUNK // NC):
        chunk = p * NC + c
        pltpu.sync_copy(zeros_hbm, acc.at[pl.ds(r0, RPT)])
        plsc.subcore_barrier()

        @pl.loop(0, NCH2)
        def _tc(tc):
            pltpu.sync_copy(src4_hbm.at[chunk, s, tc], sidx)
            pltpu.sync_copy(dst_hbm.at[s, tc], didx)
            pltpu.sync_copy(ew2_hbm.at[s, tc], ew2d)
            pltpu.async_copy(hs_hbm.at[sidx.at[0]], rows0, sem0)

            @pl.loop(0, NBC // 2)
            def _jj(jj):
                j0 = jj * 2
                pltpu.async_copy(hs_hbm.at[sidx.at[j0 + 1]], rows1, sem1)
                _proc(j0, rows0, sem0)
                pltpu.async_copy(hs_hbm.at[sidx.at[j0 + 2]], rows0, sem0)
                _proc(j0 + 1, rows1, sem1)

            _proc(NBC - 1, rows0, sem0)

        plsc.subcore_barrier()
        pltpu.sync_copy(acc.at[pl.ds(r0, RPT)],
                        out_hbm.at[chunk, pl.ds(r0, RPT)])
        plsc.subcore_barrier()


NB = 8
BR = NP // NB


def _tc1a_body(x_ref, p_ref, wrel_ref, wroot_ref, b_ref,
               hp_ref, sum_ref, ssq_ref, sacc, qacc):
    i = pl.program_id(0)
    rid = i * BR + lax.broadcasted_iota(jnp.int32, (BR, 1), 0)
    valid = (rid < N).astype(jnp.float32)
    aggr = p_ref[0] + p_ref[1]
    dn = (((1,), (1,)), ((), ()))
    hp = lax.dot_general(aggr, wrel_ref[...], dn,
                         preferred_element_type=jnp.float32)
    hp = hp + lax.dot_general(x_ref[...], wroot_ref[...], dn,
                              preferred_element_type=jnp.float32)
    hp = hp + b_ref[...][None, :]
    hp_ref[...] = hp

    @pl.when(i == 0)
    def _init():
        sacc[...] = jnp.zeros_like(sacc)
        qacc[...] = jnp.zeros_like(qacc)

    hpm = hp * valid
    sacc[...] += jnp.sum(hpm, axis=0, keepdims=True)
    qacc[...] += jnp.sum(hpm * hp, axis=0, keepdims=True)

    @pl.when(i == NB - 1)
    def _fin():
        sum_ref[...] = sacc[...]
        ssq_ref[...] = qacc[...]


def _tc1a(x, p, wrel, wroot, b):
    return pl.pallas_call(
        _tc1a_body,
        grid=(NB,),
        in_specs=[
            pl.BlockSpec((BR, F_IN), lambda i: (i, 0)),
            pl.BlockSpec((NC, BR, F_IN), lambda i: (0, i, 0)),
            pl.BlockSpec((H, F_IN), lambda i: (0, 0)),
            pl.BlockSpec((H, F_IN), lambda i: (0, 0)),
            pl.BlockSpec((H,), lambda i: (0,)),
        ],
        out_specs=[
            pl.BlockSpec((BR, H), lambda i: (i, 0)),
            pl.BlockSpec((1, H), lambda i: (0, 0)),
            pl.BlockSpec((1, H), lambda i: (0, 0)),
        ],
        out_shape=[
            jax.ShapeDtypeStruct((NP, H), jnp.float32),
            jax.ShapeDtypeStruct((1, H), jnp.float32),
            jax.ShapeDtypeStruct((1, H), jnp.float32),
        ],
        scratch_shapes=[
            pltpu.VMEM((1, H), jnp.float32),
            pltpu.VMEM((1, H), jnp.float32),
        ],
    )(x, p, wrel, wroot, b)


def _tc1b_body(hp_ref, sum_ref, ssq_ref, g_ref, be_ref, batch_ref,
               h_ref, hbf_ref, x1_ref, seg_acc, cnt_acc):
    i = pl.program_id(0)
    mean = sum_ref[...] * (1.0 / N)
    var = ssq_ref[...] * (1.0 / N) - mean * mean
    scale = lax.rsqrt(var + EPS) * g_ref[...][None, :]
    hn = (hp_ref[...] - mean) * scale + be_ref[...][None, :]
    h_ref[...] = hn
    hbf_ref[...] = hn.astype(jnp.bfloat16)
    PT = (lax.broadcasted_iota(jnp.int32, (BR, G), 1) == batch_ref[...]
          ).astype(jnp.float32)
    dnt = (((0,), (0,)), ((), ()))

    @pl.when(i == 0)
    def _init():
        seg_acc[...] = jnp.zeros_like(seg_acc)
        cnt_acc[...] = jnp.zeros_like(cnt_acc)

    seg_acc[...] += lax.dot_general(PT, hn, dnt,
                                    preferred_element_type=jnp.float32)
    cnt_acc[...] += lax.dot_general(PT, jnp.ones((BR, 1), jnp.float32), dnt,
                                    preferred_element_type=jnp.float32)

    @pl.when(i == NB - 1)
    def _fin():
        counts = jnp.clip(cnt_acc[...], 1.0, None)
        x1_ref[...] = seg_acc[...] / counts


def _tc1b(hp, ssum, ssq, g, be, batch):
    return pl.pallas_call(
        _tc1b_body,
        grid=(NB,),
        in_specs=[
            pl.BlockSpec((BR, H), lambda i: (i, 0)),
            pl.BlockSpec((1, H), lambda i: (0, 0)),
            pl.BlockSpec((1, H), lambda i: (0, 0)),
            pl.BlockSpec((H,), lambda i: (0,)),
            pl.BlockSpec((H,), lambda i: (0,)),
            pl.BlockSpec((BR, 1), lambda i: (i, 0)),
        ],
        out_specs=[
            pl.BlockSpec((BR, H), lambda i: (i, 0)),
            pl.BlockSpec((BR, H), lambda i: (i, 0)),
            pl.BlockSpec((G, H), lambda i: (0, 0)),
        ],
        out_shape=[
            jax.ShapeDtypeStruct((NP, H), jnp.float32),
            jax.ShapeDtypeStruct((NP, H), jnp.bfloat16),
            jax.ShapeDtypeStruct((G, H), jnp.float32),
        ],
        scratch_shapes=[
            pltpu.VMEM((G, H), jnp.float32),
            pltpu.VMEM((G, 1), jnp.float32),
        ],
    )(hp, ssum, ssq, g, be, batch)


def _tc2_body(h_ref, a2_ref, wrel_ref, wroot_ref, b_ref, g_ref, be_ref,
              batch_ref, x1_ref, out_ref, sacc, qacc, seg_acc, cnt_acc):
    i = pl.program_id(0)
    rid = i * BR + lax.broadcasted_iota(jnp.int32, (BR, 1), 0)
    valid = (rid < N).astype(jnp.float32)
    dn = (((1,), (1,)), ((), ()))
    h2 = lax.dot_general(a2_ref[...], wrel_ref[...], dn,
                         preferred_element_type=jnp.float32)
    h2 = h2 + lax.dot_general(h_ref[...], wroot_ref[...], dn,
                              preferred_element_type=jnp.float32)
    h2 = h2 + b_ref[...][None, :]
    PT = (lax.broadcasted_iota(jnp.int32, (BR, G), 1) == batch_ref[...]
          ).astype(jnp.float32)
    dnt = (((0,), (0,)), ((), ()))

    @pl.when(i == 0)
    def _init():
        sacc[...] = jnp.zeros_like(sacc)
        qacc[...] = jnp.zeros_like(qacc)
        seg_acc[...] = jnp.zeros_like(seg_acc)
        cnt_acc[...] = jnp.zeros_like(cnt_acc)

    h2m = h2 * valid
    sacc[...] += jnp.sum(h2m, axis=0, keepdims=True)
    qacc[...] += jnp.sum(h2m * h2, axis=0, keepdims=True)
    seg_acc[...] += lax.dot_general(PT, h2, dnt,
                                    preferred_element_type=jnp.float32)
    cnt_acc[...] += lax.dot_general(PT, jnp.ones((BR, 1), jnp.float32), dnt,
                                    preferred_element_type=jnp.float32)

    @pl.when(i == NB - 1)
    def _fin():
        mean = sacc[...] * (1.0 / N)
        var = qacc[...] * (1.0 / N) - mean * mean
        scale = lax.rsqrt(var + EPS) * g_ref[...][None, :]
        counts = jnp.clip(cnt_acc[...], 1.0, None)
        seg = seg_acc[...] / counts
        out_ref[...] = (seg - mean) * scale + be_ref[...][None, :] \
            + x1_ref[...]


def _tc_layer2(h, a2, wrel, wroot, b, g, be, batch, x1):
    return pl.pallas_call(
        _tc2_body,
        grid=(NB,),
        in_specs=[
            pl.BlockSpec((BR, H), lambda i: (i, 0)),
            pl.BlockSpec((BR, H), lambda i: (i, 0)),
            pl.BlockSpec((H, H), lambda i: (0, 0)),
            pl.BlockSpec((H, H), lambda i: (0, 0)),
            pl.BlockSpec((H,), lambda i: (0,)),
            pl.BlockSpec((H,), lambda i: (0,)),
            pl.BlockSpec((H,), lambda i: (0,)),
            pl.BlockSpec((BR, 1), lambda i: (i, 0)),
            pl.BlockSpec((G, H), lambda i: (0, 0)),
        ],
        out_specs=pl.BlockSpec((G, H), lambda i: (0, 0)),
        out_shape=jax.ShapeDtypeStruct((G, H), jnp.float32),
        scratch_shapes=[
            pltpu.VMEM((1, H), jnp.float32),
            pltpu.VMEM((1, H), jnp.float32),
            pltpu.VMEM((G, H), jnp.float32),
            pltpu.VMEM((G, 1), jnp.float32),
        ],
    )(h, a2, wrel, wroot, b, g, be, batch, x1)


def kernel(x, edge_index, edge_weight, batch, W1_rel, W1_root, b1, gamma1,
           beta1, W2_rel, W2_root, b2, gamma2, beta2):
    src = edge_index[0]
    dst = edge_index[1]
    zeros = jnp.zeros((RPT, F_IN), jnp.float32)

    src_w = src.reshape(NW, NCH1, NBC, KB)
    dst_w = dst.reshape(NW, NCH1, NBC, KB)
    ew_w = edge_weight.reshape(NW, NCH1, NBC, KB)
    dst_t = dst.reshape(NS, NCH2, NBC, KB)

    x_pad = jnp.pad(x, ((0, NP - N), (0, 0)))
    batch_pad = jnp.pad(batch, (0, NP - N), constant_values=G)[:, None]

    p1 = _sc_aggr1(x, src_w, dst_w, ew_w, zeros)
    hp, ssum, ssq = _tc1a(x_pad, p1, W1_rel, W1_root, b1)
    h_pad, h_bf, x1 = _tc1b(hp, ssum, ssq, gamma1, beta1, batch_pad)
    h = h_pad[:N]

    hbf_i32 = lax.bitcast_convert_type(h_bf.reshape(NP, H // 2, 2),
                                       jnp.int32)
    ew2 = _sc_ew2(hbf_i32, src_w, dst_w).reshape(-1)
    ew2_t = ew2.reshape(NS, NCH2, NBC, KB)

    hs = h.reshape(N, NCHUNK, CH).transpose(1, 0, 2).reshape(NCHUNK * N, CH)
    src4 = (src[None, :]
            + (jnp.arange(NCHUNK, dtype=jnp.int32) * N)[:, None])
    src4_t = src4.reshape(NCHUNK, NS, NCH2, NBC, KB)
    a2c = _sc_aggr2(hs, src4_t, dst_t, ew2_t, zeros)
    aggr2_pad = a2c.transpose(1, 0, 2).reshape(NP, H)

    return _tc_layer2(h_pad, aggr2_pad, W2_rel, W2_root, b2, gamma2, beta2,
                      batch_pad, x1)


# speedup vs baseline: 4.8981x; 1.0126x over previous
import functools

import jax
import jax.numpy as jnp
from jax import lax
from jax.experimental import pallas as pl
from jax.experimental.pallas import tpu as pltpu
from jax.experimental.pallas import tpu_sc as plsc

N = 10000
E = 320000
F_IN = 128
H = 512
G = 64
NCHUNK = 4
CH = H // NCHUNK
NC, NS = 2, 16
NW = NC * NS
KB = 80
NB1 = E // NW // KB
NB2 = E // NS // KB
NBC = 25
NCH1 = NB1 // NBC
NCH2 = NB2 // NBC
NP = 10240
RPT = NP // NS
EPS = 1e-5

_mesh = plsc.VectorSubcoreMesh(core_axis_name="c", subcore_axis_name="s",
                               num_cores=NC, num_subcores=NS)

_GDN = lax.GatherDimensionNumbers(offset_dims=(), collapsed_slice_dims=(0,),
                                  start_index_map=(0,))


def _shuffle(v, idx):
    return lax.gather(v, idx[:, None], _GDN, (1,),
                      mode=lax.GatherScatterMode.PROMISE_IN_BOUNDS)


def _scale_rows(buf, wrow, width):
    @pl.loop(0, KB // 16)
    def _grp(g):
        w16 = wrow[pl.ds(g * 16, 16)]
        for e in range(16):
            we = w16[e]
            ei = g * 16 + e
            for i in range(width // 16):
                sl = pl.ds(i * 16, 16)
                buf[ei, sl] = buf[ei, sl] * we


@functools.partial(
    pl.kernel,
    out_type=jax.ShapeDtypeStruct((NC, NP, F_IN), jnp.float32),
    mesh=_mesh,
    scratch_types=[
        pltpu.VMEM((NBC, KB), jnp.int32),
        pltpu.VMEM((NBC, KB), jnp.int32),
        pltpu.VMEM((NBC, KB), jnp.float32),
        pltpu.VMEM((KB, F_IN), jnp.float32),
        pltpu.VMEM((KB, F_IN), jnp.float32),
        pltpu.VMEM_SHARED((NP, F_IN), jnp.float32),
        pltpu.SemaphoreType.DMA,
        pltpu.SemaphoreType.DMA,
    ],
)
def _sc_aggr1(x_hbm, src_hbm, dst_hbm, ew_hbm, zeros_hbm, out_hbm,
              sidx, didx, ew2d, rows0, rows1, acc, sem0, sem1):
    c = lax.axis_index("c")
    s = lax.axis_index("s")
    w = c * NS + s
    r0 = s * RPT
    pltpu.sync_copy(zeros_hbm, acc.at[pl.ds(r0, RPT)])
    plsc.subcore_barrier()

    def _proc(j, buf, sem):
        pltpu.make_async_copy(x_hbm.at[pl.ds(0, KB)], buf, sem).wait()
        _scale_rows(buf, ew2d.at[j], F_IN)
        pltpu.sync_copy(buf, acc.at[didx.at[j]], add=True)

    @pl.loop(0, NCH1)
    def _tc(tc):
        pltpu.sync_copy(src_hbm.at[w, tc], sidx)
        pltpu.sync_copy(dst_hbm.at[w, tc], didx)
        pltpu.sync_copy(ew_hbm.at[w, tc], ew2d)
        pltpu.async_copy(x_hbm.at[sidx.at[0]], rows0, sem0)

        @pl.loop(0, NBC // 2)
        def _jj(jj):
            j0 = jj * 2
            pltpu.async_copy(x_hbm.at[sidx.at[j0 + 1]], rows1, sem1)
            _proc(j0, rows0, sem0)
            pltpu.async_copy(x_hbm.at[sidx.at[j0 + 2]], rows0, sem0)
            _proc(j0 + 1, rows1, sem1)

        _proc(NBC - 1, rows0, sem0)

    plsc.subcore_barrier()
    pltpu.sync_copy(acc.at[pl.ds(r0, RPT)], out_hbm.at[c, pl.ds(r0, RPT)])


@functools.partial(
    pl.kernel,
    out_type=jax.ShapeDtypeStruct((NW, NCH1, NBC, KB), jnp.float32),
    mesh=_mesh,
    scratch_types=[
        pltpu.VMEM((NBC, KB), jnp.int32),
        pltpu.VMEM((NBC, KB), jnp.int32),
        pltpu.VMEM((KB, H // 2), jnp.int32),
        pltpu.VMEM((KB, H // 2), jnp.int32),
        pltpu.VMEM((KB, H // 2), jnp.int32),
        pltpu.VMEM((KB, H // 2), jnp.int32),
        pltpu.VMEM((NBC, KB), jnp.float32),
        pltpu.SemaphoreType.DMA,
        pltpu.SemaphoreType.DMA,
        pltpu.SemaphoreType.DMA,
        pltpu.SemaphoreType.DMA,
    ],
)
def _sc_ew2(hbf_hbm, src_hbm, dst_hbm, out_hbm,
            sidx, didx, sr0, dr0, sr1, dr1, tot,
            ss0, sd0, ss1, sd1):
    c = lax.axis_index("c")
    s = lax.axis_index("s")
    w = c * NS + s

    def _issue(j, srows, drows, sems, semd):
        pltpu.async_copy(hbf_hbm.at[sidx.at[j]], srows, sems)
        pltpu.async_copy(hbf_hbm.at[didx.at[j]], drows, semd)

    def _proc(j, srows, drows, sems, semd):
        pltpu.make_async_copy(hbf_hbm.at[pl.ds(0, KB)], srows, sems).wait()
        pltpu.make_async_copy(hbf_hbm.at[pl.ds(0, KB)], drows, semd).wait()

        @pl.loop(0, KB // 16)
        def _grp(g):
            lane = lax.broadcasted_iota(jnp.int32, (16,), 0)
            tvec = jnp.zeros((16,), jnp.float32)
            msk = jnp.int32(-65536)
            for e in range(16):
                ei = g * 16 + e
                accs = [jnp.zeros((16,), jnp.float32) for _ in range(4)]
                for t in range(H // 32):
                    sl = pl.ds(t * 16, 16)
                    siv = srows[ei, sl]
                    div = drows[ei, sl]
                    qh = lax.bitcast_convert_type(siv & msk, jnp.float32) \
                        - lax.bitcast_convert_type(div & msk, jnp.float32)
                    ql = lax.bitcast_convert_type(siv << 16, jnp.float32) \
                        - lax.bitcast_convert_type(div << 16, jnp.float32)
                    k = 2 * (t % 2)
                    accs[k] = accs[k] + qh * qh
                    accs[k + 1] = accs[k + 1] + ql * ql
                a = (accs[0] + accs[1]) + (accs[2] + accs[3])
                for m in (8, 4, 2, 1):
                    a = a + _shuffle(a, lane ^ m)
                tvec = jnp.where(lane == e, a, tvec)
            tot[j, pl.ds(g * 16, 16)] = jnp.exp(tvec * (-1.0 / H))

    @pl.loop(0, NCH1)
    def _tc(tc):
        pltpu.sync_copy(src_hbm.at[w, tc], sidx)
        pltpu.sync_copy(dst_hbm.at[w, tc], didx)
        _issue(0, sr0, dr0, ss0, sd0)

        @pl.loop(0, NBC // 2)
        def _jj(jj):
            j0 = jj * 2
            _issue(j0 + 1, sr1, dr1, ss1, sd1)
            _proc(j0, sr0, dr0, ss0, sd0)
            _issue(j0 + 2, sr0, dr0, ss0, sd0)
            _proc(j0 + 1, sr1, dr1, ss1, sd1)

        _proc(NBC - 1, sr0, dr0, ss0, sd0)
        pltpu.sync_copy(tot, out_hbm.at[w, tc])


@functools.partial(
    pl.kernel,
    out_type=jax.ShapeDtypeStruct((NCHUNK, NP, CH), jnp.float32),
    mesh=_mesh,
    scratch_types=[
        pltpu.VMEM((NBC, KB), jnp.int32),
        pltpu.VMEM((NBC, KB), jnp.int32),
        pltpu.VMEM((NBC, KB), jnp.float32),
        pltpu.VMEM((KB, CH), jnp.float32),
        pltpu.VMEM((KB, CH), jnp.float32),
        pltpu.VMEM_SHARED((NP, CH), jnp.float32),
        pltpu.SemaphoreType.DMA,
        pltpu.SemaphoreType.DMA,
    ],
)
def _sc_aggr2(hs_hbm, src4_hbm, dst_hbm, ew2_hbm, zeros_hbm, out_hbm,
              sidx, didx, ew2d, rows0, rows1, acc, sem0, sem1):
    c = lax.axis_index("c")
    s = lax.axis_index("s")
    r0 = s * RPT

    def _proc(j, buf, sem):
        pltpu.make_async_copy(hs_hbm.at[pl.ds(0, KB)], buf, sem).wait()
        _scale_rows(buf, ew2d.at[j], CH)
        pltpu.sync_copy(buf, acc.at[didx.at[j]], add=True)

    for p in range(NCHUNK // NC):
        chunk = p * NC + c
        pltpu.sync_copy(zeros_hbm, acc.at[pl.ds(r0, RPT)])
        plsc.subcore_barrier()

        @pl.loop(0, NCH2)
        def _tc(tc):
            pltpu.sync_copy(src4_hbm.at[chunk, s, tc], sidx)
            pltpu.sync_copy(dst_hbm.at[s, tc], didx)
            pltpu.sync_copy(ew2_hbm.at[s, tc], ew2d)
            pltpu.async_copy(hs_hbm.at[sidx.at[0]], rows0, sem0)

            @pl.loop(0, NBC // 2)
            def _jj(jj):
                j0 = jj * 2
                pltpu.async_copy(hs_hbm.at[sidx.at[j0 + 1]], rows1, sem1)
                _proc(j0, rows0, sem0)
                pltpu.async_copy(hs_hbm.at[sidx.at[j0 + 2]], rows0, sem0)
                _proc(j0 + 1, rows1, sem1)

            _proc(NBC - 1, rows0, sem0)

        plsc.subcore_barrier()
        pltpu.sync_copy(acc.at[pl.ds(r0, RPT)],
                        out_hbm.at[chunk, pl.ds(r0, RPT)])
        plsc.subcore_barrier()


NB = 8
BR = NP // NB


def _tc1a_body(x_ref, p_ref, wrel_ref, wroot_ref, b_ref,
               hp_ref, sum_ref, ssq_ref, sacc, qacc):
    i = pl.program_id(0)
    rid = i * BR + lax.broadcasted_iota(jnp.int32, (BR, 1), 0)
    valid = (rid < N).astype(jnp.float32)
    aggr = p_ref[0] + p_ref[1]
    dn = (((1,), (1,)), ((), ()))
    hp = lax.dot_general(aggr, wrel_ref[...], dn,
                         preferred_element_type=jnp.float32)
    hp = hp + lax.dot_general(x_ref[...], wroot_ref[...], dn,
                              preferred_element_type=jnp.float32)
    hp = hp + b_ref[...][None, :]
    hp_ref[...] = hp

    @pl.when(i == 0)
    def _init():
        sacc[...] = jnp.zeros_like(sacc)
        qacc[...] = jnp.zeros_like(qacc)

    hpm = hp * valid
    sacc[...] += jnp.sum(hpm, axis=0, keepdims=True)
    qacc[...] += jnp.sum(hpm * hp, axis=0, keepdims=True)

    @pl.when(i == NB - 1)
    def _fin():
        sum_ref[...] = sacc[...]
        ssq_ref[...] = qacc[...]


def _tc1a(x, p, wrel, wroot, b):
    return pl.pallas_call(
        _tc1a_body,
        grid=(NB,),
        in_specs=[
            pl.BlockSpec((BR, F_IN), lambda i: (i, 0)),
            pl.BlockSpec((NC, BR, F_IN), lambda i: (0, i, 0)),
            pl.BlockSpec((H, F_IN), lambda i: (0, 0)),
            pl.BlockSpec((H, F_IN), lambda i: (0, 0)),
            pl.BlockSpec((H,), lambda i: (0,)),
        ],
        out_specs=[
            pl.BlockSpec((BR, H), lambda i: (i, 0)),
            pl.BlockSpec((1, H), lambda i: (0, 0)),
            pl.BlockSpec((1, H), lambda i: (0, 0)),
        ],
        out_shape=[
            jax.ShapeDtypeStruct((NP, H), jnp.float32),
            jax.ShapeDtypeStruct((1, H), jnp.float32),
            jax.ShapeDtypeStruct((1, H), jnp.float32),
        ],
        scratch_shapes=[
            pltpu.VMEM((1, H), jnp.float32),
            pltpu.VMEM((1, H), jnp.float32),
        ],
    )(x, p, wrel, wroot, b)


def _tc1b_body(hp_ref, sum_ref, ssq_ref, g_ref, be_ref, batch_ref,
               h_ref, hbf_ref, hcs_ref, x1_ref, seg_acc, cnt_acc):
    i = pl.program_id(0)
    mean = sum_ref[...] * (1.0 / N)
    var = ssq_ref[...] * (1.0 / N) - mean * mean
    scale = lax.rsqrt(var + EPS) * g_ref[...][None, :]
    hn = (hp_ref[...] - mean) * scale + be_ref[...][None, :]
    h_ref[...] = hn
    hbf_ref[...] = hn.astype(jnp.bfloat16)
    for k in range(NCHUNK):
        hcs_ref[k] = hn[:, k * CH:(k + 1) * CH]
    PT = (lax.broadcasted_iota(jnp.int32, (BR, G), 1) == batch_ref[...]
          ).astype(jnp.float32)
    dnt = (((0,), (0,)), ((), ()))

    @pl.when(i == 0)
    def _init():
        seg_acc[...] = jnp.zeros_like(seg_acc)
        cnt_acc[...] = jnp.zeros_like(cnt_acc)

    seg_acc[...] += lax.dot_general(PT, hn, dnt,
                                    preferred_element_type=jnp.float32)
    cnt_acc[...] += lax.dot_general(PT, jnp.ones((BR, 1), jnp.float32), dnt,
                                    preferred_element_type=jnp.float32)

    @pl.when(i == NB - 1)
    def _fin():
        counts = jnp.clip(cnt_acc[...], 1.0, None)
        x1_ref[...] = seg_acc[...] / counts


def _tc1b(hp, ssum, ssq, g, be, batch):
    return pl.pallas_call(
        _tc1b_body,
        grid=(NB,),
        in_specs=[
            pl.BlockSpec((BR, H), lambda i: (i, 0)),
            pl.BlockSpec((1, H), lambda i: (0, 0)),
            pl.BlockSpec((1, H), lambda i: (0, 0)),
            pl.BlockSpec((H,), lambda i: (0,)),
            pl.BlockSpec((H,), lambda i: (0,)),
            pl.BlockSpec((BR, 1), lambda i: (i, 0)),
        ],
        out_specs=[
            pl.BlockSpec((BR, H), lambda i: (i, 0)),
            pl.BlockSpec((BR, H), lambda i: (i, 0)),
            pl.BlockSpec((NCHUNK, BR, CH), lambda i: (0, i, 0)),
            pl.BlockSpec((G, H), lambda i: (0, 0)),
        ],
        out_shape=[
            jax.ShapeDtypeStruct((NP, H), jnp.float32),
            jax.ShapeDtypeStruct((NP, H), jnp.bfloat16),
            jax.ShapeDtypeStruct((NCHUNK, NP, CH), jnp.float32),
            jax.ShapeDtypeStruct((G, H), jnp.float32),
        ],
        scratch_shapes=[
            pltpu.VMEM((G, H), jnp.float32),
            pltpu.VMEM((G, 1), jnp.float32),
        ],
    )(hp, ssum, ssq, g, be, batch)


def _tc2_body(h_ref, a2_ref, wrel_ref, wroot_ref, b_ref, g_ref, be_ref,
              batch_ref, x1_ref, out_ref, sacc, qacc, seg_acc, cnt_acc):
    i = pl.program_id(0)
    rid = i * BR + lax.broadcasted_iota(jnp.int32, (BR, 1), 0)
    valid = (rid < N).astype(jnp.float32)
    dn = (((1,), (1,)), ((), ()))
    a2 = jnp.concatenate([a2_ref[k] for k in range(NCHUNK)], axis=1)
    h2 = lax.dot_general(a2, wrel_ref[...], dn,
                         preferred_element_type=jnp.float32)
    h2 = h2 + lax.dot_general(h_ref[...], wroot_ref[...], dn,
                              preferred_element_type=jnp.float32)
    h2 = h2 + b_ref[...][None, :]
    PT = (lax.broadcasted_iota(jnp.int32, (BR, G), 1) == batch_ref[...]
          ).astype(jnp.float32)
    dnt = (((0,), (0,)), ((), ()))

    @pl.when(i == 0)
    def _init():
        sacc[...] = jnp.zeros_like(sacc)
        qacc[...] = jnp.zeros_like(qacc)
        seg_acc[...] = jnp.zeros_like(seg_acc)
        cnt_acc[...] = jnp.zeros_like(cnt_acc)

    h2m = h2 * valid
    sacc[...] += jnp.sum(h2m, axis=0, keepdims=True)
    qacc[...] += jnp.sum(h2m * h2, axis=0, keepdims=True)
    seg_acc[...] += lax.dot_general(PT, h2, dnt,
                                    preferred_element_type=jnp.float32)
    cnt_acc[...] += lax.dot_general(PT, jnp.ones((BR, 1), jnp.float32), dnt,
                                    preferred_element_type=jnp.float32)

    @pl.when(i == NB - 1)
    def _fin():
        mean = sacc[...] * (1.0 / N)
        var = qacc[...] * (1.0 / N) - mean * mean
        scale = lax.rsqrt(var + EPS) * g_ref[...][None, :]
        counts = jnp.clip(cnt_acc[...], 1.0, None)
        seg = seg_acc[...] / counts
        out_ref[...] = (seg - mean) * scale + be_ref[...][None, :] \
            + x1_ref[...]


def _tc_layer2(h, a2, wrel, wroot, b, g, be, batch, x1):
    return pl.pallas_call(
        _tc2_body,
        grid=(NB,),
        in_specs=[
            pl.BlockSpec((BR, H), lambda i: (i, 0)),
            pl.BlockSpec((NCHUNK, BR, CH), lambda i: (0, i, 0)),
            pl.BlockSpec((H, H), lambda i: (0, 0)),
            pl.BlockSpec((H, H), lambda i: (0, 0)),
            pl.BlockSpec((H,), lambda i: (0,)),
            pl.BlockSpec((H,), lambda i: (0,)),
            pl.BlockSpec((H,), lambda i: (0,)),
            pl.BlockSpec((BR, 1), lambda i: (i, 0)),
            pl.BlockSpec((G, H), lambda i: (0, 0)),
        ],
        out_specs=pl.BlockSpec((G, H), lambda i: (0, 0)),
        out_shape=jax.ShapeDtypeStruct((G, H), jnp.float32),
        scratch_shapes=[
            pltpu.VMEM((1, H), jnp.float32),
            pltpu.VMEM((1, H), jnp.float32),
            pltpu.VMEM((G, H), jnp.float32),
            pltpu.VMEM((G, 1), jnp.float32),
        ],
    )(h, a2, wrel, wroot, b, g, be, batch, x1)


def kernel(x, edge_index, edge_weight, batch, W1_rel, W1_root, b1, gamma1,
           beta1, W2_rel, W2_root, b2, gamma2, beta2):
    src = edge_index[0]
    dst = edge_index[1]
    zeros = jnp.zeros((RPT, F_IN), jnp.float32)

    src_w = src.reshape(NW, NCH1, NBC, KB)
    dst_w = dst.reshape(NW, NCH1, NBC, KB)
    ew_w = edge_weight.reshape(NW, NCH1, NBC, KB)
    dst_t = dst.reshape(NS, NCH2, NBC, KB)

    x_pad = jnp.pad(x, ((0, NP - N), (0, 0)))
    batch_pad = jnp.pad(batch, (0, NP - N), constant_values=G)[:, None]

    p1 = _sc_aggr1(x, src_w, dst_w, ew_w, zeros)
    hp, ssum, ssq = _tc1a(x_pad, p1, W1_rel, W1_root, b1)
    h_pad, h_bf, h_cs, x1 = _tc1b(hp, ssum, ssq, gamma1, beta1, batch_pad)

    hbf_i32 = lax.bitcast_convert_type(h_bf.reshape(NP, H // 2, 2),
                                       jnp.int32)
    ew2 = _sc_ew2(hbf_i32, src_w, dst_w).reshape(-1)
    ew2_t = ew2.reshape(NS, NCH2, NBC, KB)

    hs = h_cs.reshape(NCHUNK * NP, CH)
    src4 = (src[None, :]
            + (jnp.arange(NCHUNK, dtype=jnp.int32) * NP)[:, None])
    src4_t = src4.reshape(NCHUNK, NS, NCH2, NBC, KB)
    a2c = _sc_aggr2(hs, src4_t, dst_t, ew2_t, zeros)

    return _tc_layer2(h_pad, a2c, W2_rel, W2_root, b2, gamma2, beta2,
                      batch_pad, x1)
